# Initial kernel scaffold; baseline (speedup 1.0000x reference)
import functools

import jax
import jax.numpy as jnp
from jax import lax
from jax.experimental import pallas as pl
from jax.experimental.pallas import tpu as pltpu
from jax.experimental.pallas import tpu_sc as plsc

_N = 10000
_E = 160000
_D = 256
_DH = 128
_NC = 2
_NS = 16
_CHUNK = 128
_CPT = -(-_E // (_NS * _CHUNK))
_EPT = _CPT * _CHUNK
_EPAD = _NS * _EPT
_NACC = 10016
_ZPT = _NACC // _NS
_OPT = _N // _NS


def _sc_body(xh, srcp, dstp, g_out, cnt_out,
             acc, cntacc, sidx, didx, gidx, rows, ones_v, czero, sem):
    c = lax.axis_index("c")
    s = lax.axis_index("s")

    def zrow(r, carry):
        for j in range(_DH // 16):
            rows[r, pl.ds(j * 16, 16)] = jnp.zeros((16,), jnp.float32)
        return carry
    lax.fori_loop(0, _CHUNK, zrow, 0)

    def orow(r, carry):
        ones_v[r, pl.ds(0, 16)] = jnp.ones((16,), jnp.float32)
        return carry
    lax.fori_loop(0, _CHUNK, orow, 0)

    def crow(r, carry):
        czero[r, pl.ds(0, 16)] = jnp.zeros((16,), jnp.float32)
        return carry
    lax.fori_loop(0, _ZPT, crow, 0)

    zbase = s * _ZPT
    for i in range(4):
        pltpu.sync_copy(rows.at[:, :], acc.at[pl.ds(zbase + i * _CHUNK, _CHUNK)])
    pltpu.sync_copy(rows.at[pl.ds(0, _ZPT - 4 * _CHUNK), :],
                    acc.at[pl.ds(zbase + 4 * _CHUNK, _ZPT - 4 * _CHUNK)])
    pltpu.sync_copy(czero.at[:, :], cntacc.at[pl.ds(zbase, _ZPT)])
    plsc.subcore_barrier()

    coff = c * _N

    def step(t, carry):
        ebase = s * _EPT + t * _CHUNK
        pltpu.sync_copy(srcp.at[pl.ds(ebase, _CHUNK)], sidx)
        pltpu.sync_copy(dstp.at[pl.ds(ebase, _CHUNK)], didx)
        for j in range(_CHUNK // 16):
            gidx[pl.ds(j * 16, 16)] = sidx[pl.ds(j * 16, 16)] + coff
        pltpu.async_copy(xh.at[gidx], rows, sem).wait()
        pltpu.sync_copy(rows, acc.at[didx], add=True)

        @pl.when(c == 0)
        def _():
            pltpu.sync_copy(ones_v, cntacc.at[didx], add=True)
        return carry

    lax.fori_loop(0, _CPT, step, 0)
    plsc.subcore_barrier()

    obase = s * _OPT
    pltpu.sync_copy(acc.at[pl.ds(obase, _OPT)], g_out.at[c, pl.ds(obase, _OPT)])

    @pl.when(c == 0)
    def _():
        pltpu.sync_copy(cntacc.at[pl.ds(obase, _OPT)],
                        cnt_out.at[pl.ds(obase, _OPT)])


_sc_call = pl.kernel(
    _sc_body,
    out_type=(
        jax.ShapeDtypeStruct((_NC, _N, _DH), jnp.float32),
        jax.ShapeDtypeStruct((_N, 16), jnp.float32),
    ),
    mesh=plsc.VectorSubcoreMesh(core_axis_name="c", subcore_axis_name="s",
                                num_cores=_NC, num_subcores=_NS),
    scratch_types=[
        pltpu.VMEM_SHARED((_NACC, _DH), jnp.float32),
        pltpu.VMEM_SHARED((_NACC, 16), jnp.float32),
        pltpu.VMEM((_CHUNK,), jnp.int32),
        pltpu.VMEM((_CHUNK,), jnp.int32),
        pltpu.VMEM((_CHUNK,), jnp.int32),
        pltpu.VMEM((_CHUNK, _DH), jnp.float32),
        pltpu.VMEM((_CHUNK, 16), jnp.float32),
        pltpu.VMEM((_ZPT, 16), jnp.float32),
        pltpu.SemaphoreType.DMA,
    ],
)


_RB = 1000
_NB = _N // _RB


def _tc1(x_ref, g_ref, cnt_ref, params_ref, w1_ref, h_ref, stats_ref):
    i = pl.program_id(0)
    c0 = cnt_ref[:, 0:1]
    emb = params_ref[0:1, :]
    k = jnp.where(c0 > 0.0,
                  x_ref[:, :] - emb * g_ref[:, :] / jnp.maximum(c0, 1.0),
                  0.0)
    h = jnp.dot(k, w1_ref[:, :], preferred_element_type=jnp.float32)
    h_ref[:, :] = h

    @pl.when(i == 0)
    def _():
        stats_ref[:, :] = jnp.zeros_like(stats_ref)

    stats_ref[0:1, :] += jnp.sum(h, axis=0, keepdims=True)
    stats_ref[1:2, :] += jnp.sum(h * h, axis=0, keepdims=True)


_tc1_call = pl.pallas_call(
    _tc1,
    grid=(_NB,),
    in_specs=[
        pl.BlockSpec((_RB, _D), lambda i: (i, 0)),
        pl.BlockSpec((_RB, _D), lambda i: (i, 0)),
        pl.BlockSpec((_RB, 16), lambda i: (i, 0)),
        pl.BlockSpec((8, _D), lambda i: (0, 0)),
        pl.BlockSpec((_D, _D), lambda i: (0, 0)),
    ],
    out_specs=[
        pl.BlockSpec((_RB, _D), lambda i: (i, 0)),
        pl.BlockSpec((8, _D), lambda i: (0, 0)),
    ],
    out_shape=[
        jax.ShapeDtypeStruct((_N, _D), jnp.float32),
        jax.ShapeDtypeStruct((8, _D), jnp.float32),
    ],
)


def _tc2(h_ref, x_ref, stats_ref, params_ref, w2_ref, o_ref):
    mu = stats_ref[0:1, :] * (1.0 / _N)
    var = stats_ref[1:2, :] * (1.0 / _N) - mu * mu
    inv = lax.rsqrt(var + 1e-5)
    gamma = params_ref[1:2, :]
    beta = params_ref[2:3, :]
    b2 = params_ref[3:4, :]
    hn = jnp.maximum((h_ref[:, :] - mu) * inv * gamma + beta, 0.0)
    o_ref[:, :] = (x_ref[:, :]
                   + jnp.dot(hn, w2_ref[:, :], preferred_element_type=jnp.float32)
                   + b2)


_tc2_call = pl.pallas_call(
    _tc2,
    grid=(_NB,),
    in_specs=[
        pl.BlockSpec((_RB, _D), lambda i: (i, 0)),
        pl.BlockSpec((_RB, _D), lambda i: (i, 0)),
        pl.BlockSpec((8, _D), lambda i: (0, 0)),
        pl.BlockSpec((8, _D), lambda i: (0, 0)),
        pl.BlockSpec((_D, _D), lambda i: (0, 0)),
    ],
    out_specs=pl.BlockSpec((_RB, _D), lambda i: (i, 0)),
    out_shape=jax.ShapeDtypeStruct((_N, _D), jnp.float32),
)


def kernel(x, edge_index, emb, W1, gamma, beta, W2, b2):
    src = edge_index[0]
    dst = edge_index[1]
    pad = _EPAD - _E
    srcp = jnp.concatenate([src, jnp.zeros((pad,), jnp.int32)])
    dstp = jnp.concatenate([dst, jnp.full((pad,), _N, jnp.int32)])
    xh = jnp.concatenate([x[:, :_DH], x[:, _DH:]], axis=0)

    g2, cnt16 = _sc_call(xh, srcp, dstp)
    g = jnp.concatenate([g2[0], g2[1]], axis=1)

    params = (jnp.zeros((8, _D), jnp.float32)
              .at[0].set(emb[0]).at[1].set(gamma)
              .at[2].set(beta).at[3].set(b2))

    h, stats = _tc1_call(x, g, cnt16, params, W1)
    feat = _tc2_call(h, x, stats, params, W2)
    return feat


# speedup vs baseline: 4.6971x; 4.6971x over previous
import functools

import jax
import jax.numpy as jnp
from jax import lax
from jax.experimental import pallas as pl
from jax.experimental.pallas import tpu as pltpu
from jax.experimental.pallas import tpu_sc as plsc

_N = 10000
_E = 160000
_D = 256
_DH = 128
_NC = 2
_NS = 16
_CHUNK = 128
_CPT = -(-_E // (_NS * _CHUNK))
_EPT = _CPT * _CHUNK
_EPAD = _NS * _EPT
_NACC = 10240
_ZPT = _NACC // _NS
_OPT = _ZPT


def _sc_body(xh, srcp, dstp, g_out, cnt_out,
             acc, cntacc, sidx, didx, gidx, rows, ones_v, czero, sem):
    c = lax.axis_index("c")
    s = lax.axis_index("s")

    def zrow(r, carry):
        for j in range(_DH // 16):
            rows[r, pl.ds(j * 16, 16)] = jnp.zeros((16,), jnp.float32)
        return carry
    lax.fori_loop(0, _CHUNK, zrow, 0)

    def orow(r, carry):
        ones_v[r, pl.ds(0, 16)] = jnp.ones((16,), jnp.float32)
        czero[r, pl.ds(0, 16)] = jnp.zeros((16,), jnp.float32)
        return carry
    lax.fori_loop(0, _CHUNK, orow, 0)

    zbase = s * _ZPT
    for i in range(_ZPT // _CHUNK):
        pltpu.sync_copy(rows.at[:, :], acc.at[pl.ds(zbase + i * _CHUNK, _CHUNK)])
        pltpu.sync_copy(czero.at[:, :],
                        cntacc.at[pl.ds(zbase + i * _CHUNK, _CHUNK)])
    plsc.subcore_barrier()

    coff = c * _N

    def step(t, carry):
        ebase = s * _EPT + t * _CHUNK
        pltpu.sync_copy(srcp.at[pl.ds(ebase, _CHUNK)], sidx)
        pltpu.sync_copy(dstp.at[pl.ds(ebase, _CHUNK)], didx)
        for j in range(_CHUNK // 16):
            sl = pl.ds(j * 16, 16)
            gidx[sl] = sidx[sl] + coff
        pltpu.async_copy(xh.at[gidx], rows, sem).wait()
        pltpu.sync_copy(rows, acc.at[didx], add=True)

        @pl.when(c == 0)
        def _():
            pltpu.sync_copy(ones_v, cntacc.at[didx], add=True)
        return carry

    lax.fori_loop(0, _CPT, step, 0)
    plsc.subcore_barrier()

    obase = s * _OPT
    pltpu.sync_copy(acc.at[pl.ds(obase, _OPT)], g_out.at[c, pl.ds(obase, _OPT)])

    @pl.when(c == 0)
    def _():
        pltpu.sync_copy(cntacc.at[pl.ds(obase, _OPT)],
                        cnt_out.at[pl.ds(obase, _OPT)])


@functools.cache
def _get_sc_call():
    return pl.kernel(
        _sc_body,
        out_type=(
            jax.ShapeDtypeStruct((_NC, _NACC, _DH), jnp.float32),
            jax.ShapeDtypeStruct((_NACC, 16), jnp.float32),
        ),
        mesh=plsc.VectorSubcoreMesh(core_axis_name="c", subcore_axis_name="s",
                                    num_cores=_NC, num_subcores=_NS),
        compiler_params=pltpu.CompilerParams(use_tc_tiling_on_sc=False),
        scratch_types=[
            pltpu.VMEM_SHARED((_NACC, _DH), jnp.float32),
            pltpu.VMEM_SHARED((_NACC, 16), jnp.float32),
            pltpu.VMEM((_CHUNK,), jnp.int32),
            pltpu.VMEM((_CHUNK,), jnp.int32),
            pltpu.VMEM((_CHUNK,), jnp.int32),
            pltpu.VMEM((_CHUNK, _DH), jnp.float32),
            pltpu.VMEM((_CHUNK, 16), jnp.float32),
            pltpu.VMEM((_CHUNK, 16), jnp.float32),
            pltpu.SemaphoreType.DMA,
        ],
    )


_RB = 1000
_NB = _N // _RB


def _tc1(x_ref, g_ref, cnt_ref, params_ref, w1_ref, h_ref, stats_ref):
    i = pl.program_id(0)
    c0 = cnt_ref[:, 0:1]
    emb = params_ref[0:1, :]
    k = jnp.where(c0 > 0.0,
                  x_ref[:, :] - emb * g_ref[:, :] / jnp.maximum(c0, 1.0),
                  0.0)
    h = jnp.dot(k, w1_ref[:, :], preferred_element_type=jnp.float32)
    h_ref[:, :] = h

    @pl.when(i == 0)
    def _():
        stats_ref[:, :] = jnp.zeros_like(stats_ref)

    stats_ref[0:1, :] += jnp.sum(h, axis=0, keepdims=True)
    stats_ref[1:2, :] += jnp.sum(h * h, axis=0, keepdims=True)


_tc1_call = pl.pallas_call(
    _tc1,
    grid=(_NB,),
    in_specs=[
        pl.BlockSpec((_RB, _D), lambda i: (i, 0)),
        pl.BlockSpec((_RB, _D), lambda i: (i, 0)),
        pl.BlockSpec((_RB, 16), lambda i: (i, 0)),
        pl.BlockSpec((8, _D), lambda i: (0, 0)),
        pl.BlockSpec((_D, _D), lambda i: (0, 0)),
    ],
    out_specs=[
        pl.BlockSpec((_RB, _D), lambda i: (i, 0)),
        pl.BlockSpec((8, _D), lambda i: (0, 0)),
    ],
    out_shape=[
        jax.ShapeDtypeStruct((_N, _D), jnp.float32),
        jax.ShapeDtypeStruct((8, _D), jnp.float32),
    ],
)


def _tc2(h_ref, x_ref, stats_ref, params_ref, w2_ref, o_ref):
    mu = stats_ref[0:1, :] * (1.0 / _N)
    var = stats_ref[1:2, :] * (1.0 / _N) - mu * mu
    inv = lax.rsqrt(var + 1e-5)
    gamma = params_ref[1:2, :]
    beta = params_ref[2:3, :]
    b2 = params_ref[3:4, :]
    hn = jnp.maximum((h_ref[:, :] - mu) * inv * gamma + beta, 0.0)
    o_ref[:, :] = (x_ref[:, :]
                   + jnp.dot(hn, w2_ref[:, :], preferred_element_type=jnp.float32)
                   + b2)


_tc2_call = pl.pallas_call(
    _tc2,
    grid=(_NB,),
    in_specs=[
        pl.BlockSpec((_RB, _D), lambda i: (i, 0)),
        pl.BlockSpec((_RB, _D), lambda i: (i, 0)),
        pl.BlockSpec((8, _D), lambda i: (0, 0)),
        pl.BlockSpec((8, _D), lambda i: (0, 0)),
        pl.BlockSpec((_D, _D), lambda i: (0, 0)),
    ],
    out_specs=pl.BlockSpec((_RB, _D), lambda i: (i, 0)),
    out_shape=jax.ShapeDtypeStruct((_N, _D), jnp.float32),
)


def kernel(x, edge_index, emb, W1, gamma, beta, W2, b2):
    src = edge_index[0]
    dst = edge_index[1]
    pad = _EPAD - _E
    srcp = jnp.concatenate([src, jnp.zeros((pad,), jnp.int32)])
    dstp = jnp.concatenate([dst, jnp.full((pad,), _N, jnp.int32)])
    xh = jnp.concatenate([x[:, :_DH], x[:, _DH:]], axis=0)

    g2, cntp = _get_sc_call()(xh, srcp, dstp)
    g = jnp.concatenate([g2[0, :_N], g2[1, :_N]], axis=1)
    cnt16 = cntp[:_N]

    params = (jnp.zeros((8, _D), jnp.float32)
              .at[0].set(emb[0]).at[1].set(gamma)
              .at[2].set(beta).at[3].set(b2))

    h, stats = _tc1_call(x, g, cnt16, params, W1)
    feat = _tc2_call(h, x, stats, params, W2)
    return feat


# speedup vs baseline: 7.9857x; 1.7001x over previous
import functools

import jax
import jax.numpy as jnp
from jax import lax
from jax.experimental import pallas as pl
from jax.experimental.pallas import tpu as pltpu
from jax.experimental.pallas import tpu_sc as plsc

_N = 10000
_E = 160000
_D = 256
_DH = 128
_NC = 2
_NS = 16
_CHUNK = 128
_CPT = 80
_EPT = _CPT * _CHUNK
_EPAD = _NS * _EPT
_NACC = 10240
_ZPT = _NACC // _NS
_OPT = _ZPT


def _sc_body(xh, srcp, dstp, g_out, cnt_out,
             acc, cntacc, didx0, didx1, gidx0, gidx1, rows0, rows1,
             ones_v, czero, sem0, sem1):
    c = lax.axis_index("c")
    s = lax.axis_index("s")

    def zrow(r, carry):
        for j in range(_DH // 16):
            rows0[r, pl.ds(j * 16, 16)] = jnp.zeros((16,), jnp.float32)
        return carry
    lax.fori_loop(0, _CHUNK, zrow, 0)

    def orow(r, carry):
        ones_v[r, pl.ds(0, 16)] = jnp.ones((16,), jnp.float32)
        czero[r, pl.ds(0, 16)] = jnp.zeros((16,), jnp.float32)
        return carry
    lax.fori_loop(0, _CHUNK, orow, 0)

    zbase = s * _ZPT
    for i in range(_ZPT // _CHUNK):
        pltpu.sync_copy(rows0.at[:, :], acc.at[pl.ds(zbase + i * _CHUNK, _CHUNK)])
        pltpu.sync_copy(czero.at[:, :],
                        cntacc.at[pl.ds(zbase + i * _CHUNK, _CHUNK)])
    plsc.subcore_barrier()

    coff = c * _N
    tbase = s * _EPT

    def load_idx(t, gidx, didx):
        ebase = tbase + t * _CHUNK
        pltpu.sync_copy(srcp.at[pl.ds(ebase, _CHUNK)], gidx)
        pltpu.sync_copy(dstp.at[pl.ds(ebase, _CHUNK)], didx)
        for j in range(_CHUNK // 16):
            sl = pl.ds(j * 16, 16)
            gidx[sl] = gidx[sl] + coff

    def scatter(rows, didx):
        pltpu.sync_copy(rows, acc.at[didx], add=True)

        @pl.when(c == 0)
        def _():
            pltpu.sync_copy(ones_v, cntacc.at[didx], add=True)

    load_idx(0, gidx0, didx0)
    pltpu.async_copy(xh.at[gidx0], rows0, sem0)

    def step(i, carry):
        t0 = 2 * i
        load_idx(t0 + 1, gidx1, didx1)
        pltpu.make_async_copy(xh.at[gidx0], rows0, sem0).wait()
        pltpu.async_copy(xh.at[gidx1], rows1, sem1)
        scatter(rows0, didx0)

        @pl.when(i < _CPT // 2 - 1)
        def _():
            load_idx(t0 + 2, gidx0, didx0)

        pltpu.make_async_copy(xh.at[gidx1], rows1, sem1).wait()

        @pl.when(i < _CPT // 2 - 1)
        def _():
            pltpu.async_copy(xh.at[gidx0], rows0, sem0)

        scatter(rows1, didx1)
        return carry

    lax.fori_loop(0, _CPT // 2, step, 0)
    plsc.subcore_barrier()

    obase = s * _OPT
    pltpu.sync_copy(acc.at[pl.ds(obase, _OPT)], g_out.at[c, pl.ds(obase, _OPT)])

    @pl.when(c == 0)
    def _():
        pltpu.sync_copy(cntacc.at[pl.ds(obase, _OPT)],
                        cnt_out.at[pl.ds(obase, _OPT)])


@functools.cache
def _get_sc_call():
    return pl.kernel(
        _sc_body,
        out_type=(
            jax.ShapeDtypeStruct((_NC, _NACC, _DH), jnp.float32),
            jax.ShapeDtypeStruct((_NACC, 16), jnp.float32),
        ),
        mesh=plsc.VectorSubcoreMesh(core_axis_name="c", subcore_axis_name="s",
                                    num_cores=_NC, num_subcores=_NS),
        compiler_params=pltpu.CompilerParams(use_tc_tiling_on_sc=False),
        scratch_types=[
            pltpu.VMEM_SHARED((_NACC, _DH), jnp.float32),
            pltpu.VMEM_SHARED((_NACC, 16), jnp.float32),
            pltpu.VMEM((_CHUNK,), jnp.int32),
            pltpu.VMEM((_CHUNK,), jnp.int32),
            pltpu.VMEM((_CHUNK,), jnp.int32),
            pltpu.VMEM((_CHUNK,), jnp.int32),
            pltpu.VMEM((_CHUNK, _DH), jnp.float32),
            pltpu.VMEM((_CHUNK, _DH), jnp.float32),
            pltpu.VMEM((_CHUNK, 16), jnp.float32),
            pltpu.VMEM((_CHUNK, 16), jnp.float32),
            pltpu.SemaphoreType.DMA,
            pltpu.SemaphoreType.DMA,
        ],
    )


_RB = 1000
_NB = _N // _RB


def _tc1(x_ref, g_ref, cnt_ref, params_ref, w1_ref, h_ref, stats_ref):
    i = pl.program_id(0)
    c0 = cnt_ref[:, 0:1]
    emb = params_ref[0:1, :]
    k = jnp.where(c0 > 0.0,
                  x_ref[:, :] - emb * g_ref[:, :] / jnp.maximum(c0, 1.0),
                  0.0)
    h = jnp.dot(k, w1_ref[:, :], preferred_element_type=jnp.float32)
    h_ref[:, :] = h

    @pl.when(i == 0)
    def _():
        stats_ref[:, :] = jnp.zeros_like(stats_ref)

    stats_ref[0:1, :] += jnp.sum(h, axis=0, keepdims=True)
    stats_ref[1:2, :] += jnp.sum(h * h, axis=0, keepdims=True)


_tc1_call = pl.pallas_call(
    _tc1,
    grid=(_NB,),
    in_specs=[
        pl.BlockSpec((_RB, _D), lambda i: (i, 0)),
        pl.BlockSpec((_RB, _D), lambda i: (i, 0)),
        pl.BlockSpec((_RB, 16), lambda i: (i, 0)),
        pl.BlockSpec((8, _D), lambda i: (0, 0)),
        pl.BlockSpec((_D, _D), lambda i: (0, 0)),
    ],
    out_specs=[
        pl.BlockSpec((_RB, _D), lambda i: (i, 0)),
        pl.BlockSpec((8, _D), lambda i: (0, 0)),
    ],
    out_shape=[
        jax.ShapeDtypeStruct((_N, _D), jnp.float32),
        jax.ShapeDtypeStruct((8, _D), jnp.float32),
    ],
)


def _tc2(h_ref, x_ref, stats_ref, params_ref, w2_ref, o_ref):
    mu = stats_ref[0:1, :] * (1.0 / _N)
    var = stats_ref[1:2, :] * (1.0 / _N) - mu * mu
    inv = lax.rsqrt(var + 1e-5)
    gamma = params_ref[1:2, :]
    beta = params_ref[2:3, :]
    b2 = params_ref[3:4, :]
    hn = jnp.maximum((h_ref[:, :] - mu) * inv * gamma + beta, 0.0)
    o_ref[:, :] = (x_ref[:, :]
                   + jnp.dot(hn, w2_ref[:, :], preferred_element_type=jnp.float32)
                   + b2)


_tc2_call = pl.pallas_call(
    _tc2,
    grid=(_NB,),
    in_specs=[
        pl.BlockSpec((_RB, _D), lambda i: (i, 0)),
        pl.BlockSpec((_RB, _D), lambda i: (i, 0)),
        pl.BlockSpec((8, _D), lambda i: (0, 0)),
        pl.BlockSpec((8, _D), lambda i: (0, 0)),
        pl.BlockSpec((_D, _D), lambda i: (0, 0)),
    ],
    out_specs=pl.BlockSpec((_RB, _D), lambda i: (i, 0)),
    out_shape=jax.ShapeDtypeStruct((_N, _D), jnp.float32),
)


def kernel(x, edge_index, emb, W1, gamma, beta, W2, b2):
    src = edge_index[0]
    dst = edge_index[1]
    pad = _EPAD - _E
    r = jnp.arange(pad, dtype=jnp.int32)
    srcp = jnp.concatenate([src, r % _N])
    dstp = jnp.concatenate([dst, _N + r % (_NACC - _N)])
    xh = jnp.concatenate([x[:, :_DH], x[:, _DH:]], axis=0)

    g2, cntp = _get_sc_call()(xh, srcp, dstp)
    g = jnp.concatenate([g2[0, :_N], g2[1, :_N]], axis=1)
    cnt16 = cntp[:_N]

    params = (jnp.zeros((8, _D), jnp.float32)
              .at[0].set(emb[0]).at[1].set(gamma)
              .at[2].set(beta).at[3].set(b2))

    h, stats = _tc1_call(x, g, cnt16, params, W1)
    feat = _tc2_call(h, x, stats, params, W2)
    return feat


# speedup vs baseline: 8.1157x; 1.0163x over previous
import functools

import jax
import jax.numpy as jnp
from jax import lax
from jax.experimental import pallas as pl
from jax.experimental.pallas import tpu as pltpu
from jax.experimental.pallas import tpu_sc as plsc

_N = 10000
_E = 160000
_D = 256
_DH = 128
_NC = 2
_NS = 16
_CHUNK = 128
_CPT = 80
_EPT = _CPT * _CHUNK
_EPAD = _NS * _EPT
_NACC = 10240
_ZPT = _NACC // _NS
_OPT = _ZPT


def _sc_body(xh, srcp, dstp, g_out, cnt_out,
             acc, cntacc, didx0, didx1, gidx0, gidx1, rows0, rows1,
             ones_v, czero, sem0, sem1):
    c = lax.axis_index("c")
    s = lax.axis_index("s")

    def zrow(r, carry):
        for j in range(_DH // 16):
            rows0[r, pl.ds(j * 16, 16)] = jnp.zeros((16,), jnp.float32)
        return carry
    lax.fori_loop(0, _CHUNK, zrow, 0)

    def orow(r, carry):
        ones_v[r, pl.ds(0, 16)] = jnp.ones((16,), jnp.float32)
        czero[r, pl.ds(0, 16)] = jnp.zeros((16,), jnp.float32)
        return carry
    lax.fori_loop(0, _CHUNK, orow, 0)

    zbase = s * _ZPT
    for i in range(_ZPT // _CHUNK):
        pltpu.sync_copy(rows0.at[:, :], acc.at[pl.ds(zbase + i * _CHUNK, _CHUNK)])
        pltpu.sync_copy(czero.at[:, :],
                        cntacc.at[pl.ds(zbase + i * _CHUNK, _CHUNK)])
    plsc.subcore_barrier()

    tbase = s * _EPT

    def load_idx(t, gidx, didx):
        ebase = tbase + t * _CHUNK
        pltpu.sync_copy(srcp.at[pl.ds(ebase, _CHUNK)], gidx)
        pltpu.sync_copy(dstp.at[pl.ds(ebase, _CHUNK)], didx)
        for j in range(_CHUNK // 16):
            sl = pl.ds(j * 16, 16)
            v = gidx[sl]
            gidx[sl] = v + v + c

    def scatter(rows, didx):
        pltpu.sync_copy(rows, acc.at[didx], add=True)

        @pl.when(c == 0)
        def _():
            pltpu.sync_copy(ones_v, cntacc.at[didx], add=True)

    load_idx(0, gidx0, didx0)
    pltpu.async_copy(xh.at[gidx0], rows0, sem0)

    def step(i, carry):
        t0 = 2 * i
        load_idx(t0 + 1, gidx1, didx1)
        pltpu.make_async_copy(xh.at[gidx0], rows0, sem0).wait()
        pltpu.async_copy(xh.at[gidx1], rows1, sem1)
        scatter(rows0, didx0)

        @pl.when(i < _CPT // 2 - 1)
        def _():
            load_idx(t0 + 2, gidx0, didx0)

        pltpu.make_async_copy(xh.at[gidx1], rows1, sem1).wait()

        @pl.when(i < _CPT // 2 - 1)
        def _():
            pltpu.async_copy(xh.at[gidx0], rows0, sem0)

        scatter(rows1, didx1)
        return carry

    lax.fori_loop(0, _CPT // 2, step, 0)
    plsc.subcore_barrier()

    obase = s * _OPT
    pltpu.sync_copy(acc.at[pl.ds(obase, _OPT)],
                    g_out.at[pl.ds(obase, _OPT), c])

    @pl.when(c == 0)
    def _():
        pltpu.sync_copy(cntacc.at[pl.ds(obase, _OPT)],
                        cnt_out.at[pl.ds(obase, _OPT)])


@functools.cache
def _get_sc_call():
    return pl.kernel(
        _sc_body,
        out_type=(
            jax.ShapeDtypeStruct((_NACC, _NC, _DH), jnp.float32),
            jax.ShapeDtypeStruct((_NACC, 16), jnp.float32),
        ),
        mesh=plsc.VectorSubcoreMesh(core_axis_name="c", subcore_axis_name="s",
                                    num_cores=_NC, num_subcores=_NS),
        compiler_params=pltpu.CompilerParams(use_tc_tiling_on_sc=False),
        scratch_types=[
            pltpu.VMEM_SHARED((_NACC, _DH), jnp.float32),
            pltpu.VMEM_SHARED((_NACC, 16), jnp.float32),
            pltpu.VMEM((_CHUNK,), jnp.int32),
            pltpu.VMEM((_CHUNK,), jnp.int32),
            pltpu.VMEM((_CHUNK,), jnp.int32),
            pltpu.VMEM((_CHUNK,), jnp.int32),
            pltpu.VMEM((_CHUNK, _DH), jnp.float32),
            pltpu.VMEM((_CHUNK, _DH), jnp.float32),
            pltpu.VMEM((_CHUNK, 16), jnp.float32),
            pltpu.VMEM((_CHUNK, 16), jnp.float32),
            pltpu.SemaphoreType.DMA,
            pltpu.SemaphoreType.DMA,
        ],
    )


_RB = 1000
_NB = _N // _RB


def _tc1(x_ref, g_ref, cnt_ref, params_ref, w1_ref, h_ref, stats_ref):
    i = pl.program_id(0)
    c0 = cnt_ref[:, 0:1]
    emb = params_ref[0:1, :]
    k = jnp.where(c0 > 0.0,
                  x_ref[:, :] - emb * g_ref[:, :] / jnp.maximum(c0, 1.0),
                  0.0)
    h = jnp.dot(k, w1_ref[:, :], preferred_element_type=jnp.float32)
    h_ref[:, :] = h

    @pl.when(i == 0)
    def _():
        stats_ref[:, :] = jnp.zeros_like(stats_ref)

    stats_ref[0:1, :] += jnp.sum(h, axis=0, keepdims=True)
    stats_ref[1:2, :] += jnp.sum(h * h, axis=0, keepdims=True)


_tc1_call = pl.pallas_call(
    _tc1,
    grid=(_NB,),
    in_specs=[
        pl.BlockSpec((_RB, _D), lambda i: (i, 0)),
        pl.BlockSpec((_RB, _D), lambda i: (i, 0)),
        pl.BlockSpec((_RB, 16), lambda i: (i, 0)),
        pl.BlockSpec((8, _D), lambda i: (0, 0)),
        pl.BlockSpec((_D, _D), lambda i: (0, 0)),
    ],
    out_specs=[
        pl.BlockSpec((_RB, _D), lambda i: (i, 0)),
        pl.BlockSpec((8, _D), lambda i: (0, 0)),
    ],
    out_shape=[
        jax.ShapeDtypeStruct((_N, _D), jnp.float32),
        jax.ShapeDtypeStruct((8, _D), jnp.float32),
    ],
)


def _tc2(h_ref, x_ref, stats_ref, params_ref, w2_ref, o_ref):
    mu = stats_ref[0:1, :] * (1.0 / _N)
    var = stats_ref[1:2, :] * (1.0 / _N) - mu * mu
    inv = lax.rsqrt(var + 1e-5)
    gamma = params_ref[1:2, :]
    beta = params_ref[2:3, :]
    b2 = params_ref[3:4, :]
    hn = jnp.maximum((h_ref[:, :] - mu) * inv * gamma + beta, 0.0)
    o_ref[:, :] = (x_ref[:, :]
                   + jnp.dot(hn, w2_ref[:, :], preferred_element_type=jnp.float32)
                   + b2)


_tc2_call = pl.pallas_call(
    _tc2,
    grid=(_NB,),
    in_specs=[
        pl.BlockSpec((_RB, _D), lambda i: (i, 0)),
        pl.BlockSpec((_RB, _D), lambda i: (i, 0)),
        pl.BlockSpec((8, _D), lambda i: (0, 0)),
        pl.BlockSpec((8, _D), lambda i: (0, 0)),
        pl.BlockSpec((_D, _D), lambda i: (0, 0)),
    ],
    out_specs=pl.BlockSpec((_RB, _D), lambda i: (i, 0)),
    out_shape=jax.ShapeDtypeStruct((_N, _D), jnp.float32),
)


def kernel(x, edge_index, emb, W1, gamma, beta, W2, b2):
    src = edge_index[0]
    dst = edge_index[1]
    pad = _EPAD - _E
    r = jnp.arange(pad, dtype=jnp.int32)
    srcp = jnp.concatenate([src, r % _N])
    dstp = jnp.concatenate([dst, _N + r % (_NACC - _N)])
    xh = x.reshape(2 * _N, _DH)

    g2, cntp = _get_sc_call()(xh, srcp, dstp)
    g = g2.reshape(_NACC, _D)[:_N]
    cnt16 = cntp[:_N]

    params = (jnp.zeros((8, _D), jnp.float32)
              .at[0].set(emb[0]).at[1].set(gamma)
              .at[2].set(beta).at[3].set(b2))

    h, stats = _tc1_call(x, g, cnt16, params, W1)
    feat = _tc2_call(h, x, stats, params, W2)
    return feat


# speedup vs baseline: 8.5287x; 1.0509x over previous
import functools

import jax
import jax.numpy as jnp
from jax import lax
from jax.experimental import pallas as pl
from jax.experimental.pallas import tpu as pltpu
from jax.experimental.pallas import tpu_sc as plsc

_N = 10000
_E = 160000
_D = 256
_DH = 128
_NC = 2
_NS = 16
_CHUNK = 128
_CPT = 80
_EPT = _CPT * _CHUNK
_EPAD = _NS * _EPT
_NACC = 10240
_ZPT = _NACC // _NS
_OPT = _ZPT


def _sc_body(xh, srcp, dstp, g_out, cnt_out,
             acc, cntacc, didx0, didx1, gidx0, gidx1, rows0, rows1,
             ones_v, czero, sem0, sem1):
    c = lax.axis_index("c")
    s = lax.axis_index("s")

    def zrow(r, carry):
        for j in range(_DH // 16):
            rows0[r, pl.ds(j * 16, 16)] = jnp.zeros((16,), jnp.float32)
        return carry
    lax.fori_loop(0, _CHUNK, zrow, 0)

    def orow(r, carry):
        ones_v[r, pl.ds(0, 16)] = jnp.ones((16,), jnp.float32)
        czero[r, pl.ds(0, 16)] = jnp.zeros((16,), jnp.float32)
        return carry
    lax.fori_loop(0, _CHUNK, orow, 0)

    zbase = s * _ZPT
    for i in range(_ZPT // _CHUNK):
        pltpu.sync_copy(rows0.at[:, :], acc.at[pl.ds(zbase + i * _CHUNK, _CHUNK)])
        pltpu.sync_copy(czero.at[:, :],
                        cntacc.at[pl.ds(zbase + i * _CHUNK, _CHUNK)])
    plsc.subcore_barrier()

    tbase = s * _EPT

    def load_idx(t, gidx, didx):
        ebase = tbase + t * _CHUNK
        pltpu.sync_copy(srcp.at[pl.ds(ebase, _CHUNK)], gidx)
        pltpu.sync_copy(dstp.at[pl.ds(ebase, _CHUNK)], didx)
        for j in range(_CHUNK // 16):
            sl = pl.ds(j * 16, 16)
            v = gidx[sl]
            gidx[sl] = v + v + c

    def scatter(rows, didx, cnt_core):
        pltpu.sync_copy(rows, acc.at[didx], add=True)

        @pl.when(c == cnt_core)
        def _():
            pltpu.sync_copy(ones_v, cntacc.at[didx], add=True)

    load_idx(0, gidx0, didx0)
    pltpu.async_copy(xh.at[gidx0], rows0, sem0)

    def step(i, carry):
        t0 = 2 * i
        load_idx(t0 + 1, gidx1, didx1)
        pltpu.make_async_copy(xh.at[gidx0], rows0, sem0).wait()
        pltpu.async_copy(xh.at[gidx1], rows1, sem1)
        scatter(rows0, didx0, 0)

        @pl.when(i < _CPT // 2 - 1)
        def _():
            load_idx(t0 + 2, gidx0, didx0)

        pltpu.make_async_copy(xh.at[gidx1], rows1, sem1).wait()

        @pl.when(i < _CPT // 2 - 1)
        def _():
            pltpu.async_copy(xh.at[gidx0], rows0, sem0)

        scatter(rows1, didx1, 1)
        return carry

    lax.fori_loop(0, _CPT // 2, step, 0)
    plsc.subcore_barrier()

    obase = s * _OPT
    pltpu.sync_copy(acc.at[pl.ds(obase, _OPT)],
                    g_out.at[pl.ds(obase, _OPT), c])

    pltpu.sync_copy(cntacc.at[pl.ds(obase, _OPT)],
                    cnt_out.at[c, pl.ds(obase, _OPT)])


@functools.cache
def _get_sc_call():
    return pl.kernel(
        _sc_body,
        out_type=(
            jax.ShapeDtypeStruct((_NACC, _NC, _DH), jnp.float32),
            jax.ShapeDtypeStruct((_NC, _NACC, 16), jnp.float32),
        ),
        mesh=plsc.VectorSubcoreMesh(core_axis_name="c", subcore_axis_name="s",
                                    num_cores=_NC, num_subcores=_NS),
        compiler_params=pltpu.CompilerParams(use_tc_tiling_on_sc=False),
        scratch_types=[
            pltpu.VMEM_SHARED((_NACC, _DH), jnp.float32),
            pltpu.VMEM_SHARED((_NACC, 16), jnp.float32),
            pltpu.VMEM((_CHUNK,), jnp.int32),
            pltpu.VMEM((_CHUNK,), jnp.int32),
            pltpu.VMEM((_CHUNK,), jnp.int32),
            pltpu.VMEM((_CHUNK,), jnp.int32),
            pltpu.VMEM((_CHUNK, _DH), jnp.float32),
            pltpu.VMEM((_CHUNK, _DH), jnp.float32),
            pltpu.VMEM((_CHUNK, 16), jnp.float32),
            pltpu.VMEM((_CHUNK, 16), jnp.float32),
            pltpu.SemaphoreType.DMA,
            pltpu.SemaphoreType.DMA,
        ],
    )


_RB = 1000
_NB = _N // _RB


def _tcf(x_ref, g_ref, cnt_ref, params_ref, w1_ref, w2_ref, o_ref,
         h_ref, stats_ref):
    p = pl.program_id(0)
    i = pl.program_id(1)

    @pl.when(p == 0)
    def _():
        c0 = cnt_ref[0, :, 0:1] + cnt_ref[1, :, 0:1]
        emb = params_ref[0:1, :]
        k = jnp.where(c0 > 0.0,
                      x_ref[:, :] - emb * g_ref[:, :] / jnp.maximum(c0, 1.0),
                      0.0)
        h = jnp.dot(k, w1_ref[:, :], preferred_element_type=jnp.float32)
        h_ref[pl.ds(i * _RB, _RB), :] = h

        @pl.when(i == 0)
        def _():
            stats_ref[:, :] = jnp.zeros_like(stats_ref)

        stats_ref[0:1, :] += jnp.sum(h, axis=0, keepdims=True)
        stats_ref[1:2, :] += jnp.sum(h * h, axis=0, keepdims=True)

    @pl.when(p == 1)
    def _():
        mu = stats_ref[0:1, :] * (1.0 / _N)
        var = stats_ref[1:2, :] * (1.0 / _N) - mu * mu
        inv = lax.rsqrt(var + 1e-5)
        gamma = params_ref[1:2, :]
        beta = params_ref[2:3, :]
        b2 = params_ref[3:4, :]
        h = h_ref[pl.ds(i * _RB, _RB), :]
        hn = jnp.maximum((h - mu) * inv * gamma + beta, 0.0)
        o_ref[:, :] = (x_ref[:, :]
                       + jnp.dot(hn, w2_ref[:, :],
                                 preferred_element_type=jnp.float32)
                       + b2)


_tcf_call = pl.pallas_call(
    _tcf,
    grid=(2, _NB),
    in_specs=[
        pl.BlockSpec((_RB, _D), lambda p, i: (i, 0)),
        pl.BlockSpec((_RB, _D), lambda p, i: (i * (1 - p), 0)),
        pl.BlockSpec((2, _RB, 16), lambda p, i: (0, i * (1 - p), 0)),
        pl.BlockSpec((8, _D), lambda p, i: (0, 0)),
        pl.BlockSpec((_D, _D), lambda p, i: (0, 0)),
        pl.BlockSpec((_D, _D), lambda p, i: (0, 0)),
    ],
    out_specs=pl.BlockSpec((_RB, _D), lambda p, i: (i, 0)),
    out_shape=jax.ShapeDtypeStruct((_N, _D), jnp.float32),
    scratch_shapes=[
        pltpu.VMEM((_N, _D), jnp.float32),
        pltpu.VMEM((8, _D), jnp.float32),
    ],
)


def kernel(x, edge_index, emb, W1, gamma, beta, W2, b2):
    src = edge_index[0]
    dst = edge_index[1]
    pad = _EPAD - _E
    r = jnp.arange(pad, dtype=jnp.int32)
    srcp = jnp.concatenate([src, r % _N])
    dstp = jnp.concatenate([dst, _N + r % (_NACC - _N)])
    xh = x.reshape(2 * _N, _DH)

    g2, cntp = _get_sc_call()(xh, srcp, dstp)
    g = g2.reshape(_NACC, _D)

    params = (jnp.zeros((8, _D), jnp.float32)
              .at[0].set(emb[0]).at[1].set(gamma)
              .at[2].set(beta).at[3].set(b2))

    return _tcf_call(x, g, cntp, params, W1, W2)


# speedup vs baseline: 9.6259x; 1.1287x over previous
import functools

import jax
import jax.numpy as jnp
from jax import lax
from jax.experimental import pallas as pl
from jax.experimental.pallas import tpu as pltpu
from jax.experimental.pallas import tpu_sc as plsc

_N = 10000
_E = 160000
_D = 256
_DH = 128
_NC = 2
_NS = 16
_CHUNK = 128
_CPT = 80
_EPT = _CPT * _CHUNK
_EPAD = _NS * _EPT
_NACC = 10240
_ZPT = _NACC // _NS
_OPT = _ZPT


def _sc_body(xh, epack, g_out, cnt_out,
             acc, cntacc, eidx0, eidx1, didx0, didx1, gidx0, gidx1,
             rows0, rows1, ones_v, czero, semg0, semg1, semi0, semi1):
    c = lax.axis_index("c")
    s = lax.axis_index("s")

    def zrow(r, carry):
        for j in range(_DH // 16):
            rows0[r, pl.ds(j * 16, 16)] = jnp.zeros((16,), jnp.float32)
        return carry
    lax.fori_loop(0, _CHUNK, zrow, 0)

    def orow(r, carry):
        ones_v[r, pl.ds(0, 16)] = jnp.ones((16,), jnp.float32)
        czero[r, pl.ds(0, 16)] = jnp.zeros((16,), jnp.float32)
        return carry
    lax.fori_loop(0, _CHUNK, orow, 0)

    zbase = s * _ZPT
    for i in range(_ZPT // _CHUNK):
        pltpu.sync_copy(rows0.at[:, :], acc.at[pl.ds(zbase + i * _CHUNK, _CHUNK)])
        pltpu.sync_copy(czero.at[:, :],
                        cntacc.at[pl.ds(zbase + i * _CHUNK, _CHUNK)])
    plsc.subcore_barrier()

    ebase = s * _CPT * 2 * _CHUNK

    def start_idx(t, eidx, semi):
        pltpu.async_copy(epack.at[pl.ds(ebase + t * 2 * _CHUNK, 2 * _CHUNK)],
                         eidx, semi)

    def wait_idx(eidx, semi):
        pltpu.make_async_copy(epack.at[pl.ds(ebase, 2 * _CHUNK)], eidx,
                              semi).wait()

    def build(eidx, gidx, didx):
        for j in range(_CHUNK // 16):
            sl = pl.ds(j * 16, 16)
            v = eidx[sl]
            gidx[sl] = v + v + c
            didx[sl] = eidx[pl.ds(_CHUNK + j * 16, 16)]

    def scatter(rows, didx, cnt_core):
        pltpu.sync_copy(rows, acc.at[didx], add=True)

        @pl.when(c == cnt_core)
        def _():
            pltpu.sync_copy(ones_v, cntacc.at[didx], add=True)

    start_idx(0, eidx0, semi0)
    wait_idx(eidx0, semi0)
    build(eidx0, gidx0, didx0)
    pltpu.async_copy(xh.at[gidx0], rows0, semg0)
    start_idx(1, eidx1, semi1)

    def step(i, carry):
        t0 = 2 * i
        last = _CPT // 2 - 1
        wait_idx(eidx1, semi1)
        build(eidx1, gidx1, didx1)
        pltpu.make_async_copy(xh.at[gidx0], rows0, semg0).wait()
        pltpu.async_copy(xh.at[gidx1], rows1, semg1)

        @pl.when(i < last)
        def _():
            start_idx(t0 + 2, eidx0, semi0)

        scatter(rows0, didx0, 0)

        @pl.when(i < last)
        def _():
            wait_idx(eidx0, semi0)
            build(eidx0, gidx0, didx0)

        pltpu.make_async_copy(xh.at[gidx1], rows1, semg1).wait()

        @pl.when(i < last)
        def _():
            pltpu.async_copy(xh.at[gidx0], rows0, semg0)
            start_idx(t0 + 3, eidx1, semi1)

        scatter(rows1, didx1, 1)
        return carry

    lax.fori_loop(0, _CPT // 2, step, 0)
    plsc.subcore_barrier()

    obase = s * _OPT
    pltpu.sync_copy(acc.at[pl.ds(obase, _OPT)],
                    g_out.at[pl.ds(obase, _OPT), c])

    pltpu.sync_copy(cntacc.at[pl.ds(obase, _OPT)],
                    cnt_out.at[c, pl.ds(obase, _OPT)])


@functools.cache
def _get_sc_call():
    return pl.kernel(
        _sc_body,
        out_type=(
            jax.ShapeDtypeStruct((_NACC, _NC, _DH), jnp.float32),
            jax.ShapeDtypeStruct((_NC, _NACC, 16), jnp.float32),
        ),
        mesh=plsc.VectorSubcoreMesh(core_axis_name="c", subcore_axis_name="s",
                                    num_cores=_NC, num_subcores=_NS),
        compiler_params=pltpu.CompilerParams(use_tc_tiling_on_sc=False),
        scratch_types=[
            pltpu.VMEM_SHARED((_NACC, _DH), jnp.float32),
            pltpu.VMEM_SHARED((_NACC, 16), jnp.float32),
            pltpu.VMEM((2 * _CHUNK,), jnp.int32),
            pltpu.VMEM((2 * _CHUNK,), jnp.int32),
            pltpu.VMEM((_CHUNK,), jnp.int32),
            pltpu.VMEM((_CHUNK,), jnp.int32),
            pltpu.VMEM((_CHUNK,), jnp.int32),
            pltpu.VMEM((_CHUNK,), jnp.int32),
            pltpu.VMEM((_CHUNK, _DH), jnp.float32),
            pltpu.VMEM((_CHUNK, _DH), jnp.float32),
            pltpu.VMEM((_CHUNK, 16), jnp.float32),
            pltpu.VMEM((_CHUNK, 16), jnp.float32),
            pltpu.SemaphoreType.DMA,
            pltpu.SemaphoreType.DMA,
            pltpu.SemaphoreType.DMA,
            pltpu.SemaphoreType.DMA,
        ],
    )


_RB = 1000
_NB = _N // _RB


def _tcf(x_ref, g_ref, cnt_ref, params_ref, w1_ref, w2_ref, o_ref,
         h_ref, stats_ref):
    p = pl.program_id(0)
    i = pl.program_id(1)

    @pl.when(p == 0)
    def _():
        c0 = cnt_ref[0, :, 0:1] + cnt_ref[1, :, 0:1]
        emb = params_ref[0:1, :]
        k = jnp.where(c0 > 0.0,
                      x_ref[:, :] - emb * g_ref[:, :] / jnp.maximum(c0, 1.0),
                      0.0)
        h = jnp.dot(k, w1_ref[:, :], preferred_element_type=jnp.float32)
        h_ref[pl.ds(i * _RB, _RB), :] = h

        @pl.when(i == 0)
        def _():
            stats_ref[:, :] = jnp.zeros_like(stats_ref)

        stats_ref[0:1, :] += jnp.sum(h, axis=0, keepdims=True)
        stats_ref[1:2, :] += jnp.sum(h * h, axis=0, keepdims=True)

    @pl.when(p == 1)
    def _():
        mu = stats_ref[0:1, :] * (1.0 / _N)
        var = stats_ref[1:2, :] * (1.0 / _N) - mu * mu
        inv = lax.rsqrt(var + 1e-5)
        gamma = params_ref[1:2, :]
        beta = params_ref[2:3, :]
        b2 = params_ref[3:4, :]
        h = h_ref[pl.ds(i * _RB, _RB), :]
        hn = jnp.maximum((h - mu) * inv * gamma + beta, 0.0)
        o_ref[:, :] = (x_ref[:, :]
                       + jnp.dot(hn, w2_ref[:, :],
                                 preferred_element_type=jnp.float32)
                       + b2)


_tcf_call = pl.pallas_call(
    _tcf,
    grid=(2, _NB),
    in_specs=[
        pl.BlockSpec((_RB, _D), lambda p, i: (i, 0)),
        pl.BlockSpec((_RB, _D), lambda p, i: (i * (1 - p), 0)),
        pl.BlockSpec((2, _RB, 16), lambda p, i: (0, i * (1 - p), 0)),
        pl.BlockSpec((8, _D), lambda p, i: (0, 0)),
        pl.BlockSpec((_D, _D), lambda p, i: (0, 0)),
        pl.BlockSpec((_D, _D), lambda p, i: (0, 0)),
    ],
    out_specs=pl.BlockSpec((_RB, _D), lambda p, i: (i, 0)),
    out_shape=jax.ShapeDtypeStruct((_N, _D), jnp.float32),
    scratch_shapes=[
        pltpu.VMEM((_N, _D), jnp.float32),
        pltpu.VMEM((8, _D), jnp.float32),
    ],
)


def kernel(x, edge_index, emb, W1, gamma, beta, W2, b2):
    src = edge_index[0]
    dst = edge_index[1]
    pad = _EPAD - _E
    r = jnp.arange(pad, dtype=jnp.int32)
    srcp = jnp.concatenate([src, r % _N])
    dstp = jnp.concatenate([dst, _N + r % (_NACC - _N)])
    epack = jnp.concatenate([srcp.reshape(-1, _CHUNK),
                             dstp.reshape(-1, _CHUNK)], axis=1).reshape(-1)
    xh = x.reshape(2 * _N, _DH)

    g2, cntp = _get_sc_call()(xh, epack)
    g = g2.reshape(_NACC, _D)

    params = (jnp.zeros((8, _D), jnp.float32)
              .at[0].set(emb[0]).at[1].set(gamma)
              .at[2].set(beta).at[3].set(b2))

    return _tcf_call(x, g, cntp, params, W1, W2)


# speedup vs baseline: 11.1380x; 1.1571x over previous
import functools

import jax
import jax.numpy as jnp
from jax import lax
from jax.experimental import pallas as pl
from jax.experimental.pallas import tpu as pltpu
from jax.experimental.pallas import tpu_sc as plsc

_N = 10000
_E = 160000
_D = 256
_DH = 128
_NC = 2
_NS = 16
_CHUNK = 128
_CPT = 80
_EPT = _CPT * _CHUNK
_EPAD = _NS * _EPT
_NACC = 10240
_ZPT = _NACC // _NS
_OPT = _ZPT


def _sc_body(xh, epack, g_out, cnt_out,
             acc, cntacc, eidx0, eidx1, didx0, didx1, gidx0, gidx1,
             rows0, rows1, ones_v, czero, semg0, semg1, semi0, semi1):
    c = lax.axis_index("c")
    s = lax.axis_index("s")

    def zrow(r, carry):
        for j in range(_DH // 32):
            rows0[r, pl.ds(j * 32, 32)] = jnp.zeros((32,), jnp.bfloat16)
        return carry
    lax.fori_loop(0, _CHUNK, zrow, 0)

    def orow(r, carry):
        ones_v[r, pl.ds(0, 16)] = jnp.ones((16,), jnp.float32)
        czero[r, pl.ds(0, 16)] = jnp.zeros((16,), jnp.float32)
        return carry
    lax.fori_loop(0, _CHUNK, orow, 0)

    zbase = s * _ZPT
    for i in range(_ZPT // _CHUNK):
        pltpu.sync_copy(rows0.at[:, :], acc.at[pl.ds(zbase + i * _CHUNK, _CHUNK)])
        pltpu.sync_copy(czero.at[:, :],
                        cntacc.at[pl.ds(zbase + i * _CHUNK, _CHUNK)])
    plsc.subcore_barrier()

    ebase = s * _CPT * 2 * _CHUNK

    def start_idx(t, eidx, semi):
        pltpu.async_copy(epack.at[pl.ds(ebase + t * 2 * _CHUNK, 2 * _CHUNK)],
                         eidx, semi)

    def wait_idx(eidx, semi):
        pltpu.make_async_copy(epack.at[pl.ds(ebase, 2 * _CHUNK)], eidx,
                              semi).wait()

    def build(eidx, gidx, didx):
        for j in range(_CHUNK // 16):
            sl = pl.ds(j * 16, 16)
            v = eidx[sl]
            gidx[sl] = v + v + c
            didx[sl] = eidx[pl.ds(_CHUNK + j * 16, 16)]

    def scatter(rows, didx, cnt_core):
        pltpu.sync_copy(rows, acc.at[didx], add=True)

        @pl.when(c == cnt_core)
        def _():
            pltpu.sync_copy(ones_v, cntacc.at[didx], add=True)

    start_idx(0, eidx0, semi0)
    wait_idx(eidx0, semi0)
    build(eidx0, gidx0, didx0)
    pltpu.async_copy(xh.at[gidx0], rows0, semg0)
    start_idx(1, eidx1, semi1)

    def step(i, carry):
        t0 = 2 * i
        last = _CPT // 2 - 1
        wait_idx(eidx1, semi1)
        build(eidx1, gidx1, didx1)
        pltpu.make_async_copy(xh.at[gidx0], rows0, semg0).wait()
        pltpu.async_copy(xh.at[gidx1], rows1, semg1)

        @pl.when(i < last)
        def _():
            start_idx(t0 + 2, eidx0, semi0)

        scatter(rows0, didx0, 0)

        @pl.when(i < last)
        def _():
            wait_idx(eidx0, semi0)
            build(eidx0, gidx0, didx0)

        pltpu.make_async_copy(xh.at[gidx1], rows1, semg1).wait()

        @pl.when(i < last)
        def _():
            pltpu.async_copy(xh.at[gidx0], rows0, semg0)
            start_idx(t0 + 3, eidx1, semi1)

        scatter(rows1, didx1, 1)
        return carry

    lax.fori_loop(0, _CPT // 2, step, 0)
    plsc.subcore_barrier()

    obase = s * _OPT
    pltpu.sync_copy(acc.at[pl.ds(obase, _OPT)],
                    g_out.at[pl.ds(obase, _OPT), c])

    pltpu.sync_copy(cntacc.at[pl.ds(obase, _OPT)],
                    cnt_out.at[c, pl.ds(obase, _OPT)])


@functools.cache
def _get_sc_call():
    return pl.kernel(
        _sc_body,
        out_type=(
            jax.ShapeDtypeStruct((_NACC, _NC, _DH), jnp.bfloat16),
            jax.ShapeDtypeStruct((_NC, _NACC, 16), jnp.float32),
        ),
        mesh=plsc.VectorSubcoreMesh(core_axis_name="c", subcore_axis_name="s",
                                    num_cores=_NC, num_subcores=_NS),
        compiler_params=pltpu.CompilerParams(use_tc_tiling_on_sc=False),
        scratch_types=[
            pltpu.VMEM_SHARED((_NACC, _DH), jnp.bfloat16),
            pltpu.VMEM_SHARED((_NACC, 16), jnp.float32),
            pltpu.VMEM((2 * _CHUNK,), jnp.int32),
            pltpu.VMEM((2 * _CHUNK,), jnp.int32),
            pltpu.VMEM((_CHUNK,), jnp.int32),
            pltpu.VMEM((_CHUNK,), jnp.int32),
            pltpu.VMEM((_CHUNK,), jnp.int32),
            pltpu.VMEM((_CHUNK,), jnp.int32),
            pltpu.VMEM((_CHUNK, _DH), jnp.bfloat16),
            pltpu.VMEM((_CHUNK, _DH), jnp.bfloat16),
            pltpu.VMEM((_CHUNK, 16), jnp.float32),
            pltpu.VMEM((_CHUNK, 16), jnp.float32),
            pltpu.SemaphoreType.DMA,
            pltpu.SemaphoreType.DMA,
            pltpu.SemaphoreType.DMA,
            pltpu.SemaphoreType.DMA,
        ],
    )


_RB = 1000
_NB = _N // _RB


def _tcf(x_ref, g_ref, cnt_ref, params_ref, w1_ref, w2_ref, o_ref,
         h_ref, stats_ref):
    p = pl.program_id(0)
    i = pl.program_id(1)

    @pl.when(p == 0)
    def _():
        c0 = cnt_ref[0, :, 0:1] + cnt_ref[1, :, 0:1]
        emb = params_ref[0:1, :]
        g32 = g_ref[:, :].astype(jnp.float32)
        k = jnp.where(c0 > 0.0,
                      x_ref[:, :] - emb * g32 / jnp.maximum(c0, 1.0),
                      0.0)
        h = jnp.dot(k, w1_ref[:, :], preferred_element_type=jnp.float32)
        h_ref[pl.ds(i * _RB, _RB), :] = h

        @pl.when(i == 0)
        def _():
            stats_ref[:, :] = jnp.zeros_like(stats_ref)

        stats_ref[0:1, :] += jnp.sum(h, axis=0, keepdims=True)
        stats_ref[1:2, :] += jnp.sum(h * h, axis=0, keepdims=True)

    @pl.when(p == 1)
    def _():
        mu = stats_ref[0:1, :] * (1.0 / _N)
        var = stats_ref[1:2, :] * (1.0 / _N) - mu * mu
        inv = lax.rsqrt(var + 1e-5)
        gamma = params_ref[1:2, :]
        beta = params_ref[2:3, :]
        b2 = params_ref[3:4, :]
        h = h_ref[pl.ds(i * _RB, _RB), :]
        hn = jnp.maximum((h - mu) * inv * gamma + beta, 0.0)
        o_ref[:, :] = (x_ref[:, :]
                       + jnp.dot(hn, w2_ref[:, :],
                                 preferred_element_type=jnp.float32)
                       + b2)


_tcf_call = pl.pallas_call(
    _tcf,
    grid=(2, _NB),
    in_specs=[
        pl.BlockSpec((_RB, _D), lambda p, i: (i, 0)),
        pl.BlockSpec((_RB, _D), lambda p, i: (i * (1 - p), 0)),
        pl.BlockSpec((2, _RB, 16), lambda p, i: (0, i * (1 - p), 0)),
        pl.BlockSpec((8, _D), lambda p, i: (0, 0)),
        pl.BlockSpec((_D, _D), lambda p, i: (0, 0)),
        pl.BlockSpec((_D, _D), lambda p, i: (0, 0)),
    ],
    out_specs=pl.BlockSpec((_RB, _D), lambda p, i: (i, 0)),
    out_shape=jax.ShapeDtypeStruct((_N, _D), jnp.float32),
    scratch_shapes=[
        pltpu.VMEM((_N, _D), jnp.float32),
        pltpu.VMEM((8, _D), jnp.float32),
    ],
)


def kernel(x, edge_index, emb, W1, gamma, beta, W2, b2):
    src = edge_index[0]
    dst = edge_index[1]
    pad = _EPAD - _E
    r = jnp.arange(pad, dtype=jnp.int32)
    srcp = jnp.concatenate([src, r % _N])
    dstp = jnp.concatenate([dst, _N + r % (_NACC - _N)])
    epack = jnp.concatenate([srcp.reshape(-1, _CHUNK),
                             dstp.reshape(-1, _CHUNK)], axis=1).reshape(-1)
    xh = x.astype(jnp.bfloat16).reshape(2 * _N, _DH)

    g2, cntp = _get_sc_call()(xh, epack)
    g = g2.reshape(_NACC, _D)

    params = (jnp.zeros((8, _D), jnp.float32)
              .at[0].set(emb[0]).at[1].set(gamma)
              .at[2].set(beta).at[3].set(b2))

    return _tcf_call(x, g, cntp, params, W1, W2)


# speedup vs baseline: 11.6315x; 1.0443x over previous
import functools

import jax
import jax.numpy as jnp
from jax import lax
from jax.experimental import pallas as pl
from jax.experimental.pallas import tpu as pltpu
from jax.experimental.pallas import tpu_sc as plsc

_N = 10000
_E = 160000
_D = 256
_DH = 128
_NC = 2
_NS = 16
_CHUNK = 128
_CPT = 80
_EPT = _CPT * _CHUNK
_EPAD = _NS * _EPT
_NACC = 10240
_ZPT = _NACC // _NS
_OPT = _ZPT


def _sc_body(xh, epack, g_out, cnt_out,
             acc, cntacc, eidx0, eidx1, didx0, didx1, gidx0, gidx1,
             rows0, rows1, ones_v, czero, semg0, semg1, semi0, semi1):
    c = lax.axis_index("c")
    s = lax.axis_index("s")

    def zrow(r, carry):
        for j in range(_DH // 32):
            rows0[r, pl.ds(j * 32, 32)] = jnp.zeros((32,), jnp.bfloat16)
        return carry
    lax.fori_loop(0, _CHUNK, zrow, 0)

    def orow(r, carry):
        ones_v[r, pl.ds(0, 16)] = jnp.ones((16,), jnp.float32)
        czero[r, pl.ds(0, 16)] = jnp.zeros((16,), jnp.float32)
        return carry
    lax.fori_loop(0, _CHUNK, orow, 0)

    zbase = s * _ZPT
    for i in range(_ZPT // _CHUNK):
        pltpu.sync_copy(rows0.at[:, :], acc.at[pl.ds(zbase + i * _CHUNK, _CHUNK)])
        pltpu.sync_copy(czero.at[:, :],
                        cntacc.at[pl.ds(zbase + i * _CHUNK, _CHUNK)])
    plsc.subcore_barrier()

    ebase = s * _CPT * 2 * _CHUNK

    def start_idx(t, eidx, semi):
        pltpu.async_copy(epack.at[pl.ds(ebase + t * 2 * _CHUNK, 2 * _CHUNK)],
                         eidx, semi)

    def wait_idx(eidx, semi):
        pltpu.make_async_copy(epack.at[pl.ds(ebase, 2 * _CHUNK)], eidx,
                              semi).wait()

    def build(eidx, gidx, didx):
        for j in range(_CHUNK // 16):
            sl = pl.ds(j * 16, 16)
            v = eidx[sl]
            gidx[sl] = v + v + c
            didx[sl] = eidx[pl.ds(_CHUNK + j * 16, 16)]

    def scatter(rows, didx, cnt_core):
        pltpu.sync_copy(rows, acc.at[didx], add=True)

        @pl.when(c == cnt_core)
        def _():
            pltpu.sync_copy(ones_v, cntacc.at[didx], add=True)

    start_idx(0, eidx0, semi0)
    wait_idx(eidx0, semi0)
    build(eidx0, gidx0, didx0)
    pltpu.async_copy(xh.at[gidx0], rows0, semg0)
    start_idx(1, eidx1, semi1)

    def step(i, carry):
        t0 = 2 * i
        last = _CPT // 2 - 1
        wait_idx(eidx1, semi1)
        build(eidx1, gidx1, didx1)
        pltpu.make_async_copy(xh.at[gidx0], rows0, semg0).wait()
        pltpu.async_copy(xh.at[gidx1], rows1, semg1)

        @pl.when(i < last)
        def _():
            start_idx(t0 + 2, eidx0, semi0)

        scatter(rows0, didx0, 0)

        @pl.when(i < last)
        def _():
            wait_idx(eidx0, semi0)
            build(eidx0, gidx0, didx0)

        pltpu.make_async_copy(xh.at[gidx1], rows1, semg1).wait()

        @pl.when(i < last)
        def _():
            pltpu.async_copy(xh.at[gidx0], rows0, semg0)
            start_idx(t0 + 3, eidx1, semi1)

        scatter(rows1, didx1, 1)
        return carry

    lax.fori_loop(0, _CPT // 2, step, 0)
    plsc.subcore_barrier()

    obase = s * _OPT
    pltpu.sync_copy(acc.at[pl.ds(obase, _OPT)],
                    g_out.at[pl.ds(obase, _OPT), c])

    pltpu.sync_copy(cntacc.at[pl.ds(obase, _OPT)],
                    cnt_out.at[c, pl.ds(obase, _OPT)])


@functools.cache
def _get_sc_call():
    return pl.kernel(
        _sc_body,
        out_type=(
            jax.ShapeDtypeStruct((_NACC, _NC, _DH), jnp.bfloat16),
            jax.ShapeDtypeStruct((_NC, _NACC, 16), jnp.float32),
        ),
        mesh=plsc.VectorSubcoreMesh(core_axis_name="c", subcore_axis_name="s",
                                    num_cores=_NC, num_subcores=_NS),
        compiler_params=pltpu.CompilerParams(use_tc_tiling_on_sc=False),
        scratch_types=[
            pltpu.VMEM_SHARED((_NACC, _DH), jnp.bfloat16),
            pltpu.VMEM_SHARED((_NACC, 16), jnp.float32),
            pltpu.VMEM((2 * _CHUNK,), jnp.int32),
            pltpu.VMEM((2 * _CHUNK,), jnp.int32),
            pltpu.VMEM((_CHUNK,), jnp.int32),
            pltpu.VMEM((_CHUNK,), jnp.int32),
            pltpu.VMEM((_CHUNK,), jnp.int32),
            pltpu.VMEM((_CHUNK,), jnp.int32),
            pltpu.VMEM((_CHUNK, _DH), jnp.bfloat16),
            pltpu.VMEM((_CHUNK, _DH), jnp.bfloat16),
            pltpu.VMEM((_CHUNK, 16), jnp.float32),
            pltpu.VMEM((_CHUNK, 16), jnp.float32),
            pltpu.SemaphoreType.DMA,
            pltpu.SemaphoreType.DMA,
            pltpu.SemaphoreType.DMA,
            pltpu.SemaphoreType.DMA,
        ],
    )


_RB = 2000
_NB = _N // _RB


def _tcf(x_ref, g_ref, cnt_ref, params_ref, w1_ref, w2_ref, o_ref,
         h_ref, xs_ref, stats_ref):
    p = pl.program_id(0)
    i = pl.program_id(1)

    @pl.when(p == 0)
    def _():
        xb = x_ref[:, :]
        xs_ref[pl.ds(i * _RB, _RB), :] = xb
        c0 = cnt_ref[0, :, 0:1] + cnt_ref[1, :, 0:1]
        emb = params_ref[0:1, :]
        g32 = g_ref[:, :].astype(jnp.float32)
        k = jnp.where(c0 > 0.0,
                      xb - emb * g32 / jnp.maximum(c0, 1.0),
                      0.0)
        h = jnp.dot(k.astype(jnp.bfloat16), w1_ref[:, :],
                    preferred_element_type=jnp.float32)
        h_ref[pl.ds(i * _RB, _RB), :] = h

        @pl.when(i == 0)
        def _():
            stats_ref[:, :] = jnp.zeros_like(stats_ref)

        stats_ref[0:1, :] += jnp.sum(h, axis=0, keepdims=True)
        stats_ref[1:2, :] += jnp.sum(h * h, axis=0, keepdims=True)

    @pl.when(p == 1)
    def _():
        mu = stats_ref[0:1, :] * (1.0 / _N)
        var = stats_ref[1:2, :] * (1.0 / _N) - mu * mu
        inv = lax.rsqrt(var + 1e-5)
        gamma = params_ref[1:2, :]
        beta = params_ref[2:3, :]
        b2 = params_ref[3:4, :]
        h = h_ref[pl.ds(i * _RB, _RB), :]
        hn = jnp.maximum((h - mu) * inv * gamma + beta, 0.0)
        o_ref[:, :] = (xs_ref[pl.ds(i * _RB, _RB), :]
                       + jnp.dot(hn.astype(jnp.bfloat16), w2_ref[:, :],
                                 preferred_element_type=jnp.float32)
                       + b2)


_tcf_call = pl.pallas_call(
    _tcf,
    grid=(2, _NB),
    in_specs=[
        pl.BlockSpec((_RB, _D), lambda p, i: (i * (1 - p), 0)),
        pl.BlockSpec((_RB, _D), lambda p, i: (i * (1 - p), 0)),
        pl.BlockSpec((2, _RB, 16), lambda p, i: (0, i * (1 - p), 0)),
        pl.BlockSpec((8, _D), lambda p, i: (0, 0)),
        pl.BlockSpec((_D, _D), lambda p, i: (0, 0)),
        pl.BlockSpec((_D, _D), lambda p, i: (0, 0)),
    ],
    out_specs=pl.BlockSpec((_RB, _D), lambda p, i: (i, 0)),
    out_shape=jax.ShapeDtypeStruct((_N, _D), jnp.float32),
    scratch_shapes=[
        pltpu.VMEM((_N, _D), jnp.float32),
        pltpu.VMEM((_N, _D), jnp.float32),
        pltpu.VMEM((8, _D), jnp.float32),
    ],
)


def kernel(x, edge_index, emb, W1, gamma, beta, W2, b2):
    src = edge_index[0]
    dst = edge_index[1]
    pad = _EPAD - _E
    r = jnp.arange(pad, dtype=jnp.int32)
    srcp = jnp.concatenate([src, r % _N])
    dstp = jnp.concatenate([dst, _N + r % (_NACC - _N)])
    epack = jnp.concatenate([srcp.reshape(-1, _CHUNK),
                             dstp.reshape(-1, _CHUNK)], axis=1).reshape(-1)
    xh = x.astype(jnp.bfloat16).reshape(2 * _N, _DH)

    g2, cntp = _get_sc_call()(xh, epack)
    g = g2.reshape(_NACC, _D)

    params = (jnp.zeros((8, _D), jnp.float32)
              .at[0].set(emb[0]).at[1].set(gamma)
              .at[2].set(beta).at[3].set(b2))

    return _tcf_call(x, g, cntp, params,
                     W1.astype(jnp.bfloat16), W2.astype(jnp.bfloat16))


# speedup vs baseline: 11.6455x; 1.0012x over previous
import functools

import jax
import jax.numpy as jnp
from jax import lax
from jax.experimental import pallas as pl
from jax.experimental.pallas import tpu as pltpu
from jax.experimental.pallas import tpu_sc as plsc

_N = 10000
_E = 160000
_D = 256
_DH = 128
_NC = 2
_NS = 16
_CHUNK = 128
_CPT = 80
_EPT = _CPT * _CHUNK
_EPAD = _NS * _EPT
_NACC = 10240
_ZPT = _NACC // _NS
_OPT = _ZPT


def _sc_body(xh, epack, g_out, cnt_out,
             acc, cntacc, eidx0, eidx1, didx0, didx1, gidx0, gidx1,
             rows0, rows1, ones_v, czero, semg0, semg1, semi0, semi1,
             sems0, sems1, semc0, semc1):
    c = lax.axis_index("c")
    s = lax.axis_index("s")

    def zrow(r, carry):
        for j in range(_DH // 32):
            rows0[r, pl.ds(j * 32, 32)] = jnp.zeros((32,), jnp.bfloat16)
        return carry
    lax.fori_loop(0, _CHUNK, zrow, 0)

    def orow(r, carry):
        ones_v[r, pl.ds(0, 16)] = jnp.ones((16,), jnp.float32)
        czero[r, pl.ds(0, 16)] = jnp.zeros((16,), jnp.float32)
        return carry
    lax.fori_loop(0, _CHUNK, orow, 0)

    zbase = s * _ZPT
    for i in range(_ZPT // _CHUNK):
        pltpu.sync_copy(rows0.at[:, :], acc.at[pl.ds(zbase + i * _CHUNK, _CHUNK)])
        pltpu.sync_copy(czero.at[:, :],
                        cntacc.at[pl.ds(zbase + i * _CHUNK, _CHUNK)])
    plsc.subcore_barrier()

    ebase = s * _CPT * 2 * _CHUNK

    def start_idx(t, eidx, semi):
        pltpu.async_copy(epack.at[pl.ds(ebase + t * 2 * _CHUNK, 2 * _CHUNK)],
                         eidx, semi)

    def wait_idx(eidx, semi):
        pltpu.make_async_copy(epack.at[pl.ds(ebase, 2 * _CHUNK)], eidx,
                              semi).wait()

    def build(eidx, gidx, didx):
        for j in range(_CHUNK // 16):
            sl = pl.ds(j * 16, 16)
            v = eidx[sl]
            gidx[sl] = v + v + c
            didx[sl] = eidx[pl.ds(_CHUNK + j * 16, 16)]

    def scatter(rows, didx, cnt_core, sems, semc):
        pltpu.async_copy(rows, acc.at[didx], sems, add=True)

        @pl.when(c == cnt_core)
        def _():
            pltpu.async_copy(ones_v, cntacc.at[didx], semc, add=True)

    def wait_scat(rows, didx, sems):
        pltpu.make_async_copy(rows, acc.at[didx], sems).wait()

    def wait_cnt(cnt_core, didx, semc):
        @pl.when(c == cnt_core)
        def _():
            pltpu.make_async_copy(ones_v, cntacc.at[didx], semc).wait()

    start_idx(0, eidx0, semi0)
    wait_idx(eidx0, semi0)
    build(eidx0, gidx0, didx0)
    pltpu.async_copy(xh.at[gidx0], rows0, semg0)
    start_idx(1, eidx1, semi1)

    def step(i, carry):
        t0 = 2 * i
        last = _CPT // 2 - 1
        wait_idx(eidx1, semi1)

        @pl.when(i > 0)
        def _():
            wait_cnt(1, didx1, semc1)
            wait_scat(rows1, didx1, sems1)
        build(eidx1, gidx1, didx1)
        pltpu.make_async_copy(xh.at[gidx0], rows0, semg0).wait()
        pltpu.async_copy(xh.at[gidx1], rows1, semg1)

        @pl.when(i < last)
        def _():
            start_idx(t0 + 2, eidx0, semi0)

        scatter(rows0, didx0, 0, sems0, semc0)

        @pl.when(i < last)
        def _():
            wait_idx(eidx0, semi0)
            wait_cnt(0, didx0, semc0)
            wait_scat(rows0, didx0, sems0)
            build(eidx0, gidx0, didx0)

        pltpu.make_async_copy(xh.at[gidx1], rows1, semg1).wait()

        @pl.when(i < last)
        def _():
            pltpu.async_copy(xh.at[gidx0], rows0, semg0)
            start_idx(t0 + 3, eidx1, semi1)

        scatter(rows1, didx1, 1, sems1, semc1)
        return carry

    lax.fori_loop(0, _CPT // 2, step, 0)
    wait_cnt(0, didx0, semc0)
    wait_scat(rows0, didx0, sems0)
    wait_cnt(1, didx1, semc1)
    wait_scat(rows1, didx1, sems1)
    plsc.subcore_barrier()

    obase = s * _OPT
    pltpu.sync_copy(acc.at[pl.ds(obase, _OPT)],
                    g_out.at[pl.ds(obase, _OPT), c])

    pltpu.sync_copy(cntacc.at[pl.ds(obase, _OPT)],
                    cnt_out.at[c, pl.ds(obase, _OPT)])


@functools.cache
def _get_sc_call():
    return pl.kernel(
        _sc_body,
        out_type=(
            jax.ShapeDtypeStruct((_NACC, _NC, _DH), jnp.bfloat16),
            jax.ShapeDtypeStruct((_NC, _NACC, 16), jnp.float32),
        ),
        mesh=plsc.VectorSubcoreMesh(core_axis_name="c", subcore_axis_name="s",
                                    num_cores=_NC, num_subcores=_NS),
        compiler_params=pltpu.CompilerParams(use_tc_tiling_on_sc=False),
        scratch_types=[
            pltpu.VMEM_SHARED((_NACC, _DH), jnp.bfloat16),
            pltpu.VMEM_SHARED((_NACC, 16), jnp.float32),
            pltpu.VMEM((2 * _CHUNK,), jnp.int32),
            pltpu.VMEM((2 * _CHUNK,), jnp.int32),
            pltpu.VMEM((_CHUNK,), jnp.int32),
            pltpu.VMEM((_CHUNK,), jnp.int32),
            pltpu.VMEM((_CHUNK,), jnp.int32),
            pltpu.VMEM((_CHUNK,), jnp.int32),
            pltpu.VMEM((_CHUNK, _DH), jnp.bfloat16),
            pltpu.VMEM((_CHUNK, _DH), jnp.bfloat16),
            pltpu.VMEM((_CHUNK, 16), jnp.float32),
            pltpu.VMEM((_CHUNK, 16), jnp.float32),
            pltpu.SemaphoreType.DMA,
            pltpu.SemaphoreType.DMA,
            pltpu.SemaphoreType.DMA,
            pltpu.SemaphoreType.DMA,
            pltpu.SemaphoreType.DMA,
            pltpu.SemaphoreType.DMA,
            pltpu.SemaphoreType.DMA,
            pltpu.SemaphoreType.DMA,
        ],
    )


_RB = 2000
_NB = _N // _RB


def _tcf(x_ref, g_ref, cnt_ref, params_ref, w1_ref, w2_ref, o_ref,
         h_ref, xs_ref, stats_ref):
    p = pl.program_id(0)
    i = pl.program_id(1)

    @pl.when(p == 0)
    def _():
        xb = x_ref[:, :]
        xs_ref[pl.ds(i * _RB, _RB), :] = xb
        c0 = cnt_ref[0, :, 0:1] + cnt_ref[1, :, 0:1]
        emb = params_ref[0:1, :]
        g32 = g_ref[:, :].astype(jnp.float32)
        k = jnp.where(c0 > 0.0,
                      xb - emb * g32 / jnp.maximum(c0, 1.0),
                      0.0)
        h = jnp.dot(k.astype(jnp.bfloat16), w1_ref[:, :],
                    preferred_element_type=jnp.float32)
        h_ref[pl.ds(i * _RB, _RB), :] = h

        @pl.when(i == 0)
        def _():
            stats_ref[:, :] = jnp.zeros_like(stats_ref)

        stats_ref[0:1, :] += jnp.sum(h, axis=0, keepdims=True)
        stats_ref[1:2, :] += jnp.sum(h * h, axis=0, keepdims=True)

    @pl.when(p == 1)
    def _():
        mu = stats_ref[0:1, :] * (1.0 / _N)
        var = stats_ref[1:2, :] * (1.0 / _N) - mu * mu
        inv = lax.rsqrt(var + 1e-5)
        gamma = params_ref[1:2, :]
        beta = params_ref[2:3, :]
        b2 = params_ref[3:4, :]
        h = h_ref[pl.ds(i * _RB, _RB), :]
        hn = jnp.maximum((h - mu) * inv * gamma + beta, 0.0)
        o_ref[:, :] = (xs_ref[pl.ds(i * _RB, _RB), :]
                       + jnp.dot(hn.astype(jnp.bfloat16), w2_ref[:, :],
                                 preferred_element_type=jnp.float32)
                       + b2)


_tcf_call = pl.pallas_call(
    _tcf,
    grid=(2, _NB),
    in_specs=[
        pl.BlockSpec((_RB, _D), lambda p, i: (i * (1 - p), 0)),
        pl.BlockSpec((_RB, _D), lambda p, i: (i * (1 - p), 0)),
        pl.BlockSpec((2, _RB, 16), lambda p, i: (0, i * (1 - p), 0)),
        pl.BlockSpec((8, _D), lambda p, i: (0, 0)),
        pl.BlockSpec((_D, _D), lambda p, i: (0, 0)),
        pl.BlockSpec((_D, _D), lambda p, i: (0, 0)),
    ],
    out_specs=pl.BlockSpec((_RB, _D), lambda p, i: (i, 0)),
    out_shape=jax.ShapeDtypeStruct((_N, _D), jnp.float32),
    scratch_shapes=[
        pltpu.VMEM((_N, _D), jnp.float32),
        pltpu.VMEM((_N, _D), jnp.float32),
        pltpu.VMEM((8, _D), jnp.float32),
    ],
)


def kernel(x, edge_index, emb, W1, gamma, beta, W2, b2):
    src = edge_index[0]
    dst = edge_index[1]
    pad = _EPAD - _E
    r = jnp.arange(pad, dtype=jnp.int32)
    srcp = jnp.concatenate([src, r % _N])
    dstp = jnp.concatenate([dst, _N + r % (_NACC - _N)])
    epack = jnp.concatenate([srcp.reshape(-1, _CHUNK),
                             dstp.reshape(-1, _CHUNK)], axis=1).reshape(-1)
    xh = x.astype(jnp.bfloat16).reshape(2 * _N, _DH)

    g2, cntp = _get_sc_call()(xh, epack)
    g = g2.reshape(_NACC, _D)

    params = (jnp.zeros((8, _D), jnp.float32)
              .at[0].set(emb[0]).at[1].set(gamma)
              .at[2].set(beta).at[3].set(b2))

    return _tcf_call(x, g, cntp, params,
                     W1.astype(jnp.bfloat16), W2.astype(jnp.bfloat16))
